# Initial kernel scaffold; baseline (speedup 1.0000x reference)
import jax
import jax.numpy as jnp
from jax.experimental import pallas as pl


def kernel(x, edge_index, edge_weight, W1, b1, W2, b2, g1, be1, g2, be2, fc1W, fc1b, fc2W, fc2b):
    raise NotImplementedError("write your pallas kernel here")


# speedup vs baseline: 6.8480x; 6.8480x over previous
import functools

import jax
import jax.numpy as jnp
from jax import lax
from jax.experimental import pallas as pl
from jax.experimental.pallas import tpu as pltpu
from jax.experimental.pallas import tpu_sc as plsc

N = 10000
E = 320000
F = 128
NCORES = 2
NSUB = 16
NTILES = NCORES * NSUB
EPT = E // NTILES
CHUNK = 80
NCH = EPT // CHUNK
F2 = F // 2
EPS = E // NSUB
NCH2 = EPS // CHUNK
NP = 10240
RPT = NP // NSUB
ZR = 128
DEGP = 10240
DPT = DEGP // NSUB
BN_EPS = 1e-5
BLK = 2000
GRID = N // BLK

@functools.cache
def _sc_mesh():
    return plsc.VectorSubcoreMesh(core_axis_name="c", subcore_axis_name="s",
                                  num_cores=NCORES, num_subcores=NSUB)


def _deg_body(dst_hbm, w_hbm, deg_hbm, dst_v, w_v, zero_v, deg_sh):
    c = lax.axis_index("c")
    s = lax.axis_index("s")
    wid = c * NSUB + s

    def zb(i, carry):
        zero_v[pl.ds(pl.multiple_of(i * 16, 8), 16)] = jnp.zeros((16,), jnp.float32)
        return carry
    lax.fori_loop(0, DPT // 16, zb, 0)
    soff = pl.multiple_of(s * DPT, 8)
    pltpu.sync_copy(zero_v, deg_sh.at[pl.ds(soff, DPT)])
    plsc.subcore_barrier()

    pltpu.sync_copy(dst_hbm.at[wid], dst_v)
    pltpu.sync_copy(w_hbm.at[wid], w_v)

    def chunk(i, carry):
        off = pl.multiple_of(i * CHUNK, 8)
        pltpu.sync_copy(w_v.at[pl.ds(off, CHUNK)], deg_sh.at[dst_v.at[i]], add=True)
        return carry
    lax.fori_loop(0, NCH, chunk, 0)

    plsc.subcore_barrier()
    pltpu.sync_copy(deg_sh.at[pl.ds(soff, DPT)], deg_hbm.at[c, pl.ds(soff, DPT)])


@functools.cache
def _deg_kernel():
    return pl.kernel(
        _deg_body,
        out_type=jax.ShapeDtypeStruct((NCORES, DEGP), jnp.float32),
        mesh=_sc_mesh(),
        scratch_types=[
            pltpu.VMEM((NCH, CHUNK), jnp.int32),
            pltpu.VMEM((EPT,), jnp.float32),
            pltpu.VMEM((DPT,), jnp.float32),
            pltpu.VMEM_SHARED((DEGP,), jnp.float32),
        ],
    )


def _deg_call(dst3, wtile):
    return _deg_kernel()(dst3, wtile)


def _msg_body(src_hbm, dst_hbm, w_hbm, hs_hbm, acc_hbm,
              src_v, dst_v, w_v, rows_v, zrow_v, acc_sh, sem):
    c = lax.axis_index("c")
    s = lax.axis_index("s")
    wid = c * NSUB + s

    def zb(i, carry):
        for j in range(F2 // 16):
            zrow_v[i, pl.ds(j * 16, 16)] = jnp.zeros((16,), jnp.float32)
        return carry
    lax.fori_loop(0, ZR, zb, 0)
    for k in range(RPT // ZR):
        pltpu.sync_copy(zrow_v, acc_sh.at[pl.ds(s * RPT + k * ZR, ZR)])
    plsc.subcore_barrier()

    pltpu.sync_copy(src_hbm.at[wid], src_v)
    pltpu.sync_copy(dst_hbm.at[s], dst_v)
    pltpu.sync_copy(w_hbm.at[s], w_v)

    def chunk(i, carry):
        off = pl.multiple_of(i * CHUNK, 8)
        pltpu.async_copy(hs_hbm.at[src_v.at[pl.ds(off, CHUNK)]], rows_v, sem).wait()

        def scale(g, carry2):
            woff = pl.multiple_of(off + g * 16, 8)
            wv = w_v[pl.ds(woff, 16)]
            for e in range(16):
                er = g * 16 + e
                cval = wv[e]
                for j in range(F2 // 16):
                    sl = pl.ds(j * 16, 16)
                    rows_v[er, sl] = rows_v[er, sl] * cval
            return carry2
        lax.fori_loop(0, CHUNK // 16, scale, 0)

        pltpu.sync_copy(rows_v, acc_sh.at[dst_v.at[i]], add=True)
        return carry
    lax.fori_loop(0, NCH2, chunk, 0)

    plsc.subcore_barrier()
    pltpu.sync_copy(acc_sh.at[pl.ds(s * RPT, RPT)], acc_hbm.at[c, pl.ds(s * RPT, RPT)])


@functools.cache
def _msg_kernel():
    return pl.kernel(
        _msg_body,
        out_type=jax.ShapeDtypeStruct((NCORES, NP, F2), jnp.float32),
        mesh=_sc_mesh(),
        compiler_params=pltpu.CompilerParams(use_tc_tiling_on_sc=False),
        scratch_types=[
            pltpu.VMEM((EPS,), jnp.int32),
            pltpu.VMEM((NCH2, CHUNK), jnp.int32),
            pltpu.VMEM((EPS,), jnp.float32),
            pltpu.VMEM((CHUNK, F2), jnp.float32),
            pltpu.VMEM((ZR, F2), jnp.float32),
            pltpu.VMEM_SHARED((NP, F2), jnp.float32),
            pltpu.SemaphoreType.DMA,
        ],
    )


def _msg_call(src, dst3, wtile, hs):
    return _msg_kernel()(src, dst3, wtile, hs)


def _prep_body(deg_ref, x_ref, w_ref, h_ref, hs_ref, dis_ref):
    deg = deg_ref[:, 0:1] + deg_ref[:, 1:2] + 1.0
    dis = lax.rsqrt(deg)
    h = jnp.dot(x_ref[...], w_ref[...], preferred_element_type=jnp.float32)
    h_ref[...] = h
    hs_ref[...] = h * dis
    dis_ref[...] = dis


def _prep_call(degT, x, W):
    return pl.pallas_call(
        _prep_body,
        grid=(GRID,),
        in_specs=[
            pl.BlockSpec((BLK, 2), lambda i: (i, 0)),
            pl.BlockSpec((BLK, F), lambda i: (i, 0)),
            pl.BlockSpec((F, F), lambda i: (0, 0)),
        ],
        out_specs=[
            pl.BlockSpec((BLK, F), lambda i: (i, 0)),
            pl.BlockSpec((BLK, F), lambda i: (i, 0)),
            pl.BlockSpec((BLK, 1), lambda i: (i, 0)),
        ],
        out_shape=[
            jax.ShapeDtypeStruct((N, F), jnp.float32),
            jax.ShapeDtypeStruct((N, F), jnp.float32),
            jax.ShapeDtypeStruct((N, 1), jnp.float32),
        ],
    )(degT, x, W)


def _post_body(acc0_ref, acc1_ref, h_ref, dis_ref, b_ref, r_ref, sum_ref, sq_ref):
    i = pl.program_id(0)
    dis = dis_ref[...]
    acc = jnp.concatenate([acc0_ref[...], acc1_ref[...]], axis=1)
    pre = dis * acc + (dis * dis) * h_ref[...] + b_ref[...]
    r = jnp.maximum(pre, 0.0)
    r_ref[...] = r

    @pl.when(i == 0)
    def _():
        sum_ref[...] = jnp.zeros_like(sum_ref)
        sq_ref[...] = jnp.zeros_like(sq_ref)

    sum_ref[...] += jnp.sum(r, axis=0, keepdims=True)
    sq_ref[...] += jnp.sum(r * r, axis=0, keepdims=True)


def _post_call(acc0, acc1, h, dis, b):
    return pl.pallas_call(
        _post_body,
        grid=(GRID,),
        in_specs=[
            pl.BlockSpec((BLK, F2), lambda i: (i, 0)),
            pl.BlockSpec((BLK, F2), lambda i: (i, 0)),
            pl.BlockSpec((BLK, F), lambda i: (i, 0)),
            pl.BlockSpec((BLK, 1), lambda i: (i, 0)),
            pl.BlockSpec((1, F), lambda i: (0, 0)),
        ],
        out_specs=[
            pl.BlockSpec((BLK, F), lambda i: (i, 0)),
            pl.BlockSpec((1, F), lambda i: (0, 0)),
            pl.BlockSpec((1, F), lambda i: (0, 0)),
        ],
        out_shape=[
            jax.ShapeDtypeStruct((N, F), jnp.float32),
            jax.ShapeDtypeStruct((1, F), jnp.float32),
            jax.ShapeDtypeStruct((1, F), jnp.float32),
        ],
    )(acc0, acc1, h, dis, b)


def _bnmm_body(r_ref, sum_ref, sq_ref, g_ref, be_ref, w_ref, dis_ref,
               hn_ref, h2_ref, hs2_ref):
    mu = sum_ref[...] * (1.0 / N)
    var = sq_ref[...] * (1.0 / N) - mu * mu
    inv = g_ref[...] * lax.rsqrt(var + BN_EPS)
    hn = (r_ref[...] - mu) * inv + be_ref[...]
    hn_ref[...] = hn
    h2 = jnp.dot(hn, w_ref[...], preferred_element_type=jnp.float32)
    h2_ref[...] = h2
    hs2_ref[...] = h2 * dis_ref[...]


def _bnmm_call(r, sm, sq, g, be, W, dis):
    return pl.pallas_call(
        _bnmm_body,
        grid=(GRID,),
        in_specs=[
            pl.BlockSpec((BLK, F), lambda i: (i, 0)),
            pl.BlockSpec((1, F), lambda i: (0, 0)),
            pl.BlockSpec((1, F), lambda i: (0, 0)),
            pl.BlockSpec((1, F), lambda i: (0, 0)),
            pl.BlockSpec((1, F), lambda i: (0, 0)),
            pl.BlockSpec((F, F), lambda i: (0, 0)),
            pl.BlockSpec((BLK, 1), lambda i: (i, 0)),
        ],
        out_specs=[
            pl.BlockSpec((BLK, F), lambda i: (i, 0)),
            pl.BlockSpec((BLK, F), lambda i: (i, 0)),
            pl.BlockSpec((BLK, F), lambda i: (i, 0)),
        ],
        out_shape=[
            jax.ShapeDtypeStruct((N, F), jnp.float32),
            jax.ShapeDtypeStruct((N, F), jnp.float32),
            jax.ShapeDtypeStruct((N, F), jnp.float32),
        ],
    )(r, sm, sq, g, be, W, dis)


def _final_body(r_ref, sum_ref, sq_ref, g_ref, be_ref, x_ref, h1_ref,
                w1x_ref, w1a_ref, w1b_ref, b1_ref, w2_ref, b2_ref, o_ref):
    mu = sum_ref[...] * (1.0 / N)
    var = sq_ref[...] * (1.0 / N) - mu * mu
    inv = g_ref[...] * lax.rsqrt(var + BN_EPS)
    h2n = (r_ref[...] - mu) * inv + be_ref[...]
    z = (jnp.dot(x_ref[...], w1x_ref[...], preferred_element_type=jnp.float32)
         + jnp.dot(h1_ref[...], w1a_ref[...], preferred_element_type=jnp.float32)
         + jnp.dot(h2n, w1b_ref[...], preferred_element_type=jnp.float32)
         + b1_ref[...])
    z = jnp.maximum(z, 0.0)
    o = jnp.sum(z * w2_ref[...], axis=1, keepdims=True) + b2_ref[...]
    o_ref[...] = jnp.maximum(o, 0.0)


def _final_call(r, sm, sq, g, be, x, h1, w1x, w1a, w1b, b1, w2row, b2):
    return pl.pallas_call(
        _final_body,
        grid=(GRID,),
        in_specs=[
            pl.BlockSpec((BLK, F), lambda i: (i, 0)),
            pl.BlockSpec((1, F), lambda i: (0, 0)),
            pl.BlockSpec((1, F), lambda i: (0, 0)),
            pl.BlockSpec((1, F), lambda i: (0, 0)),
            pl.BlockSpec((1, F), lambda i: (0, 0)),
            pl.BlockSpec((BLK, F), lambda i: (i, 0)),
            pl.BlockSpec((BLK, F), lambda i: (i, 0)),
            pl.BlockSpec((F, F), lambda i: (0, 0)),
            pl.BlockSpec((F, F), lambda i: (0, 0)),
            pl.BlockSpec((F, F), lambda i: (0, 0)),
            pl.BlockSpec((1, F), lambda i: (0, 0)),
            pl.BlockSpec((1, F), lambda i: (0, 0)),
            pl.BlockSpec((1, 1), lambda i: (0, 0)),
        ],
        out_specs=pl.BlockSpec((BLK, 1), lambda i: (i, 0)),
        out_shape=jax.ShapeDtypeStruct((N, 1), jnp.float32),
    )(r, sm, sq, g, be, x, h1, w1x, w1a, w1b, b1, w2row, b2)


def kernel(x, edge_index, edge_weight, W1, b1, W2, b2, g1, be1, g2, be2,
           fc1W, fc1b, fc2W, fc2b):
    src = edge_index[0].astype(jnp.int32)
    dst = edge_index[1].astype(jnp.int32)
    dst3 = dst.reshape(NTILES, NCH, CHUNK)
    wtile = edge_weight.reshape(NTILES, EPT)

    src2 = jnp.stack([src * 2, src * 2 + 1]).reshape(NTILES, EPS)
    dst3m = dst.reshape(NSUB, NCH2, CHUNK)
    wm = edge_weight.reshape(NSUB, EPS)

    deg2 = _deg_call(dst3, wtile)
    degT = deg2[:, :N].T

    h1, hs1, dis = _prep_call(degT, x, W1)
    acc1 = _msg_call(src2, dst3m, wm, hs1.reshape(2 * N, F2))[:, :N]
    r1, s1, q1 = _post_call(acc1[0], acc1[1], h1, dis, b1.reshape(1, F))
    h1n, h2, hs2 = _bnmm_call(r1, s1, q1, g1.reshape(1, F), be1.reshape(1, F),
                              W2, dis)
    acc2 = _msg_call(src2, dst3m, wm, hs2.reshape(2 * N, F2))[:, :N]
    r2, s2, q2 = _post_call(acc2[0], acc2[1], h2, dis, b2.reshape(1, F))
    o = _final_call(r2, s2, q2, g2.reshape(1, F), be2.reshape(1, F), x, h1n,
                    fc1W[:F], fc1W[F:2 * F], fc1W[2 * F:],
                    fc1b.reshape(1, F), fc2W.reshape(1, F), fc2b.reshape(1, 1))
    return o.reshape(-1)
